# Initial kernel scaffold; baseline (speedup 1.0000x reference)
import jax
import jax.numpy as jnp
from jax.experimental import pallas as pl


def kernel(x, edge_index, edge_weight, segment_ids, W1_1, W2_1, b1, W1_2, W2_2, b2, W1_3, W2_3, b3, Wd, bd):
    raise NotImplementedError("write your pallas kernel here")


# speedup vs baseline: 5.7062x; 5.7062x over previous
import functools

import jax
import jax.numpy as jnp
from jax import lax
from jax.experimental import pallas as pl
from jax.experimental.pallas import tpu as pltpu
from jax.experimental.pallas import tpu_sc as plsc

N = 50000
E = 1600000
B = 32
C = 32

NC = 2
NS = 16
NW = NC * NS
L = 16

TILE_E = E // NW
CH = 80
NCHUNK = TILE_E // CH

WR = 200

ACC1_PAD = 51200
Z1 = ACC1_PAD // NS
W1T = 10
WCH1 = N // W1T

BLK = 2000
NGRID = N // BLK

_mesh = plsc.VectorSubcoreMesh(core_axis_name="c", subcore_axis_name="s")


@functools.partial(
    pl.kernel,
    out_type=jax.ShapeDtypeStruct((NC * N,), jnp.float32),
    mesh=_mesh,
    scratch_types=[
        pltpu.VMEM((2, CH), jnp.int32),
        pltpu.VMEM((2, CH), jnp.int32),
        pltpu.VMEM((2, CH), jnp.float32),
        pltpu.VMEM((2, CH), jnp.float32),
        pltpu.VMEM((Z1,), jnp.float32),
        pltpu.VMEM((WCH1,), jnp.float32),
        pltpu.VMEM_SHARED((ACC1_PAD,), jnp.float32),
    ],
)
def _sc_scalar_agg(x_hbm, src_hbm, dst_hbm, ew_hbm, out_hbm,
                   srcb, dstb, ewb, xb, zbuf, wbuf, acc):
    cidx = lax.axis_index("c")
    sidx = lax.axis_index("s")
    zeros = jnp.zeros((L,), jnp.float32)

    @pl.loop(0, Z1 // L)
    def _zero_stage(i):
        zbuf[pl.ds(i * L, L)] = zeros

    pltpu.sync_copy(zbuf, acc.at[pl.ds(sidx * Z1, Z1)])
    plsc.subcore_barrier()

    edge0 = (cidx * NS + sidx) * TILE_E

    @pl.loop(0, NCHUNK)
    def _chunk(g):
        base = edge0 + g * CH
        pltpu.sync_copy(src_hbm.at[pl.ds(base, CH)], srcb.at[0])
        pltpu.sync_copy(dst_hbm.at[pl.ds(base, CH)], dstb.at[0])
        pltpu.sync_copy(ew_hbm.at[pl.ds(base, CH)], ewb.at[0])
        pltpu.sync_copy(x_hbm.at[srcb.at[0]], xb.at[0])
        for j in range(CH // L):
            sl = pl.ds(j * L, L)
            xb[0, sl] = xb[0, sl] * ewb[0, sl]
        pltpu.sync_copy(xb.at[0], acc.at[dstb.at[0]], add=True)

    plsc.subcore_barrier()

    @pl.when(sidx < W1T)
    def _writeout():
        pltpu.sync_copy(acc.at[pl.ds(sidx * WCH1, WCH1)], wbuf)
        pltpu.sync_copy(wbuf, out_hbm.at[pl.ds(cidx * N + sidx * WCH1, WCH1)])


@functools.partial(
    pl.kernel,
    out_type=jax.ShapeDtypeStruct((NC, N, C), jnp.float32),
    mesh=_mesh,
    scratch_types=[
        pltpu.VMEM((2, CH), jnp.int32),
        pltpu.VMEM((2, CH), jnp.int32),
        pltpu.VMEM((2, CH), jnp.float32),
        pltpu.VMEM((2, CH, C), jnp.float32),
        pltpu.VMEM((WR, C), jnp.float32),
        pltpu.VMEM_SHARED((N, C), jnp.float32),
    ],
    compiler_params=pltpu.CompilerParams(use_tc_tiling_on_sc=False),
)
def _sc_row_agg(h_hbm, src_hbm, dst_hbm, ew_hbm, out_hbm,
                srcb, dstb, ewb, rowsb, wbuf, acc):
    cidx = lax.axis_index("c")
    sidx = lax.axis_index("s")
    zeros = jnp.zeros((L,), jnp.float32)
    iota = lax.iota(jnp.int32, L)

    @pl.loop(0, WR)
    def _zero_stage(r):
        wbuf[r, pl.ds(0, L)] = zeros
        wbuf[r, pl.ds(L, L)] = zeros

    @pl.when(sidx < W1T)
    def _zero_acc():
        @pl.loop(0, WCH1 // WR)
        def _(i):
            pltpu.sync_copy(wbuf, acc.at[pl.ds(sidx * WCH1 + i * WR, WR)])

    plsc.subcore_barrier()

    edge0 = (cidx * NS + sidx) * TILE_E

    @pl.loop(0, NCHUNK)
    def _chunk(g):
        base = edge0 + g * CH
        pltpu.sync_copy(src_hbm.at[pl.ds(base, CH)], srcb.at[0])
        pltpu.sync_copy(dst_hbm.at[pl.ds(base, CH)], dstb.at[0])
        pltpu.sync_copy(ew_hbm.at[pl.ds(base, CH)], ewb.at[0])
        pltpu.sync_copy(h_hbm.at[srcb.at[0]], rowsb.at[0])

        gdn = lax.GatherDimensionNumbers(
            offset_dims=(), collapsed_slice_dims=(0,), start_index_map=(0,))

        @pl.loop(0, CH // L)
        def _scale(j):
            wv = ewb[0, pl.ds(j * L, L)]
            for el in range(L):
                w = lax.gather(wv, jnp.full((L, 1), el, jnp.int32), gdn, (1,),
                               mode=lax.GatherScatterMode.PROMISE_IN_BOUNDS)
                e = j * L + el
                rowsb[0, e, pl.ds(0, L)] = rowsb[0, e, pl.ds(0, L)] * w
                rowsb[0, e, pl.ds(L, L)] = rowsb[0, e, pl.ds(L, L)] * w

        pltpu.sync_copy(rowsb.at[0], acc.at[dstb.at[0]], add=True)

    plsc.subcore_barrier()

    @pl.when(sidx < W1T)
    def _writeout():
        @pl.loop(0, WCH1 // WR)
        def _(i):
            r0 = sidx * WCH1 + i * WR
            pltpu.sync_copy(acc.at[pl.ds(r0, WR)], wbuf)
            pltpu.sync_copy(wbuf, out_hbm.at[cidx, pl.ds(r0, WR)])


def _tc_dense1_body(sp_ref, x_ref, w1_ref, w2_ref, b_ref, o_ref):
    s = sp_ref[0] + sp_ref[1]
    h = s * w1_ref[...] + x_ref[...] * w2_ref[...] + b_ref[...]
    o_ref[...] = jnp.maximum(h, 0.0)


def _tc_dense1(s_parts, x, w1, w2, b):
    return pl.pallas_call(
        _tc_dense1_body,
        grid=(NGRID,),
        in_specs=[
            pl.BlockSpec((NC, BLK, 1), lambda i: (0, i, 0)),
            pl.BlockSpec((BLK, 1), lambda i: (i, 0)),
            pl.BlockSpec((1, C), lambda i: (0, 0)),
            pl.BlockSpec((1, C), lambda i: (0, 0)),
            pl.BlockSpec((1, C), lambda i: (0, 0)),
        ],
        out_specs=pl.BlockSpec((BLK, C), lambda i: (i, 0)),
        out_shape=jax.ShapeDtypeStruct((N, C), jnp.float32),
    )(s_parts, x, w1, w2, b)


def _tc_dense_body(a_ref, h_ref, w1_ref, w2_ref, b_ref, o_ref):
    agg = a_ref[0] + a_ref[1]
    hn = (jnp.dot(agg, w1_ref[...], preferred_element_type=jnp.float32)
          + jnp.dot(h_ref[...], w2_ref[...], preferred_element_type=jnp.float32)
          + b_ref[...])
    o_ref[...] = jnp.maximum(hn, 0.0)


def _tc_dense(a, h, w1, w2, b):
    return pl.pallas_call(
        _tc_dense_body,
        grid=(NGRID,),
        in_specs=[
            pl.BlockSpec((NC, BLK, C), lambda i: (0, i, 0)),
            pl.BlockSpec((BLK, C), lambda i: (i, 0)),
            pl.BlockSpec((C, C), lambda i: (0, 0)),
            pl.BlockSpec((C, C), lambda i: (0, 0)),
            pl.BlockSpec((1, C), lambda i: (0, 0)),
        ],
        out_specs=pl.BlockSpec((BLK, C), lambda i: (i, 0)),
        out_shape=jax.ShapeDtypeStruct((N, C), jnp.float32),
    )(a, h, w1, w2, b)


def _tc_final_body(a_ref, h_ref, w1_ref, w2_ref, b_ref, seg_ref, wd_ref,
                   bd_ref, o_ref, pool_acc, cnt_acc):
    i = pl.program_id(0)

    @pl.when(i == 0)
    def _init():
        pool_acc[...] = jnp.zeros_like(pool_acc)
        cnt_acc[...] = jnp.zeros_like(cnt_acc)

    agg = a_ref[0] + a_ref[1]
    h3 = (jnp.dot(agg, w1_ref[...], preferred_element_type=jnp.float32)
          + jnp.dot(h_ref[...], w2_ref[...], preferred_element_type=jnp.float32)
          + b_ref[...])
    h3 = jnp.maximum(h3, 0.0)

    onehot = (seg_ref[...] == lax.broadcasted_iota(jnp.int32, (BLK, B), 1))
    onehot = onehot.astype(jnp.float32)
    dn = (((0,), (0,)), ((), ()))
    pool_acc[...] += lax.dot_general(onehot, h3, dn,
                                     preferred_element_type=jnp.float32)
    cnt_acc[...] += lax.dot_general(onehot, jnp.ones((BLK, C), jnp.float32),
                                    dn, preferred_element_type=jnp.float32)

    @pl.when(i == NGRID - 1)
    def _finish():
        pooled = pool_acc[...] / jnp.maximum(cnt_acc[...], 1.0)
        z = jnp.dot(pooled, wd_ref[...],
                    preferred_element_type=jnp.float32) + bd_ref[...]
        o_ref[...] = jax.nn.sigmoid(z)


def _tc_final(a, h, w1, w2, b, seg, wd, bd):
    return pl.pallas_call(
        _tc_final_body,
        grid=(NGRID,),
        in_specs=[
            pl.BlockSpec((NC, BLK, C), lambda i: (0, i, 0)),
            pl.BlockSpec((BLK, C), lambda i: (i, 0)),
            pl.BlockSpec((C, C), lambda i: (0, 0)),
            pl.BlockSpec((C, C), lambda i: (0, 0)),
            pl.BlockSpec((1, C), lambda i: (0, 0)),
            pl.BlockSpec((BLK, 1), lambda i: (i, 0)),
            pl.BlockSpec((C, 1), lambda i: (0, 0)),
            pl.BlockSpec((1, 1), lambda i: (0, 0)),
        ],
        out_specs=pl.BlockSpec((B, 1), lambda i: (0, 0)),
        out_shape=jax.ShapeDtypeStruct((B, 1), jnp.float32),
        scratch_shapes=[
            pltpu.VMEM((B, C), jnp.float32),
            pltpu.VMEM((B, C), jnp.float32),
        ],
    )(a, h, w1, w2, b, seg, wd, bd)


def kernel(x, edge_index, edge_weight, segment_ids,
           W1_1, W2_1, b1, W1_2, W2_2, b2, W1_3, W2_3, b3, Wd, bd):
    src = edge_index[0]
    dst = edge_index[1]
    x_flat = x.reshape(N)

    s_parts = _sc_scalar_agg(x_flat, src, dst, edge_weight)
    h1 = _tc_dense1(s_parts.reshape(NC, N, 1), x,
                    W1_1, W2_1, b1.reshape(1, C))

    a2 = _sc_row_agg(h1, src, dst, edge_weight)
    h2 = _tc_dense(a2, h1, W1_2, W2_2, b2.reshape(1, C))

    a3 = _sc_row_agg(h2, src, dst, edge_weight)
    out = _tc_final(a3, h2, W1_3, W2_3, b3.reshape(1, C),
                    segment_ids.reshape(N, 1), Wd, bd.reshape(1, 1))
    return out


# speedup vs baseline: 20.3333x; 3.5634x over previous
import functools

import jax
import jax.numpy as jnp
from jax import lax
from jax.experimental import pallas as pl
from jax.experimental.pallas import tpu as pltpu
from jax.experimental.pallas import tpu_sc as plsc

N = 50000
E = 1600000
B = 32
C = 32

NC = 2
NS = 16
NW = NC * NS
L = 16

TILE_E = E // NW
CH = 80
NSUB = 5
BE = CH * NSUB
NBLK = TILE_E // BE
NSUB1 = 25
BE1 = CH * NSUB1
NBLK1 = TILE_E // BE1

WR = 200

ACC1_PAD = 51200
Z1 = ACC1_PAD // NS
W1T = 10
WCH1 = N // W1T

BLK = 2000
NGRID = N // BLK

_mesh = plsc.VectorSubcoreMesh(core_axis_name="c", subcore_axis_name="s")


@functools.partial(
    pl.kernel,
    out_type=jax.ShapeDtypeStruct((NC * N,), jnp.float32),
    mesh=_mesh,
    scratch_types=[
        pltpu.VMEM((NSUB1, CH), jnp.int32),
        pltpu.VMEM((NSUB1, CH), jnp.int32),
        pltpu.VMEM((NSUB1, CH), jnp.float32),
        pltpu.VMEM((NSUB1, CH), jnp.float32),
        pltpu.VMEM((Z1,), jnp.float32),
        pltpu.VMEM((WCH1,), jnp.float32),
        pltpu.VMEM_SHARED((ACC1_PAD,), jnp.float32),
        pltpu.SemaphoreType.DMA,
        pltpu.SemaphoreType.DMA,
        pltpu.SemaphoreType.DMA,
    ],
    compiler_params=pltpu.CompilerParams(use_tc_tiling_on_sc=False),
)
def _sc_scalar_agg(x_hbm, src_hbm, dst_hbm, ew_hbm, out_hbm,
                   srcb, dstb, ewb, xb, zbuf, wbuf, acc,
                   isem, gsem, ssem):
    cidx = lax.axis_index("c")
    sidx = lax.axis_index("s")
    zeros = jnp.zeros((L,), jnp.float32)

    @pl.loop(0, Z1 // L)
    def _zero_stage(i):
        zbuf[pl.ds(i * L, L)] = zeros

    pltpu.sync_copy(zbuf, acc.at[pl.ds(sidx * Z1, Z1)])
    plsc.subcore_barrier()

    row0 = (cidx * NS + sidx) * (TILE_E // CH)

    @pl.loop(0, NBLK1)
    def _block(g):
        rbase = row0 + g * NSUB1
        c1 = pltpu.async_copy(src_hbm.at[pl.ds(rbase, NSUB1)], srcb, isem)
        c2 = pltpu.async_copy(dst_hbm.at[pl.ds(rbase, NSUB1)], dstb, isem)
        c3 = pltpu.async_copy(ew_hbm.at[pl.ds(rbase, NSUB1)], ewb, isem)
        c1.wait()
        c2.wait()
        c3.wait()
        gathers = [
            pltpu.async_copy(x_hbm.at[srcb.at[k]], xb.at[k], gsem)
            for k in range(NSUB1)
        ]
        for cp in gathers:
            cp.wait()

        @pl.loop(0, NSUB1)
        def _mul(k):
            for j in range(CH // L):
                sl = pl.ds(j * L, L)
                xb[k, sl] = xb[k, sl] * ewb[k, sl]

        scatters = [
            pltpu.async_copy(xb.at[k], acc.at[dstb.at[k]], ssem, add=True)
            for k in range(NSUB1)
        ]
        for cp in scatters:
            cp.wait()

    plsc.subcore_barrier()

    @pl.when(sidx < W1T)
    def _writeout():
        pltpu.sync_copy(acc.at[pl.ds(sidx * WCH1, WCH1)], wbuf)
        pltpu.sync_copy(wbuf, out_hbm.at[pl.ds(cidx * N + sidx * WCH1, WCH1)])


@functools.partial(
    pl.kernel,
    out_type=jax.ShapeDtypeStruct((NC, N, C), jnp.float32),
    mesh=_mesh,
    scratch_types=[
        pltpu.VMEM((NSUB, CH), jnp.int32),
        pltpu.VMEM((NSUB, CH), jnp.int32),
        pltpu.VMEM((NSUB, CH), jnp.float32),
        pltpu.VMEM((BE, C), jnp.float32),
        pltpu.VMEM((WR, C), jnp.float32),
        pltpu.VMEM_SHARED((N, C), jnp.float32),
        pltpu.SemaphoreType.DMA,
        pltpu.SemaphoreType.DMA,
        pltpu.SemaphoreType.DMA,
    ],
    compiler_params=pltpu.CompilerParams(use_tc_tiling_on_sc=False),
)
def _sc_row_agg(h_hbm, src_hbm, dst_hbm, ew_hbm, out_hbm,
                srcb, dstb, ewb, rowsb, wbuf, acc, isem, gsem, ssem):
    cidx = lax.axis_index("c")
    sidx = lax.axis_index("s")
    zeros = jnp.zeros((L,), jnp.float32)
    iota = lax.iota(jnp.int32, L)

    @pl.loop(0, WR)
    def _zero_stage(r):
        wbuf[r, pl.ds(0, L)] = zeros
        wbuf[r, pl.ds(L, L)] = zeros

    @pl.when(sidx < W1T)
    def _zero_acc():
        @pl.loop(0, WCH1 // WR)
        def _(i):
            pltpu.sync_copy(wbuf, acc.at[pl.ds(sidx * WCH1 + i * WR, WR)])

    plsc.subcore_barrier()

    row0 = (cidx * NS + sidx) * (TILE_E // CH)
    gdn = lax.GatherDimensionNumbers(
        offset_dims=(), collapsed_slice_dims=(0,), start_index_map=(0,))

    @pl.loop(0, NBLK)
    def _block(g):
        rbase = row0 + g * NSUB
        c1 = pltpu.async_copy(src_hbm.at[pl.ds(rbase, NSUB)], srcb, isem)
        c2 = pltpu.async_copy(dst_hbm.at[pl.ds(rbase, NSUB)], dstb, isem)
        c3 = pltpu.async_copy(ew_hbm.at[pl.ds(rbase, NSUB)], ewb, isem)
        c1.wait()
        c2.wait()
        c3.wait()
        gathers = [
            pltpu.async_copy(h_hbm.at[srcb.at[k]],
                             rowsb.at[pl.ds(k * CH, CH)], gsem)
            for k in range(NSUB)
        ]
        for cp in gathers:
            cp.wait()

        @pl.loop(0, NSUB)
        def _scale(k):
            for jj in range(CH // L):
                wv = ewb[k, pl.ds(jj * L, L)]
                for el in range(L):
                    w = lax.gather(wv, jnp.full((L, 1), el, jnp.int32), gdn,
                                   (1,),
                                   mode=lax.GatherScatterMode.PROMISE_IN_BOUNDS)
                    e = k * CH + jj * L + el
                    rowsb[e, pl.ds(0, L)] = rowsb[e, pl.ds(0, L)] * w
                    rowsb[e, pl.ds(L, L)] = rowsb[e, pl.ds(L, L)] * w

        scatters = [
            pltpu.async_copy(rowsb.at[pl.ds(k * CH, CH)],
                             acc.at[dstb.at[k]], ssem, add=True)
            for k in range(NSUB)
        ]
        for cp in scatters:
            cp.wait()

    plsc.subcore_barrier()

    @pl.when(sidx < W1T)
    def _writeout():
        @pl.loop(0, WCH1 // WR)
        def _(i):
            r0 = sidx * WCH1 + i * WR
            pltpu.sync_copy(acc.at[pl.ds(r0, WR)], wbuf)
            pltpu.sync_copy(wbuf, out_hbm.at[cidx, pl.ds(r0, WR)])


def _tc_dense1_body(sp_ref, x_ref, w1_ref, w2_ref, b_ref, o_ref):
    s = sp_ref[0] + sp_ref[1]
    h = s * w1_ref[...] + x_ref[...] * w2_ref[...] + b_ref[...]
    o_ref[...] = jnp.maximum(h, 0.0)


def _tc_dense1(s_parts, x, w1, w2, b):
    return pl.pallas_call(
        _tc_dense1_body,
        grid=(NGRID,),
        in_specs=[
            pl.BlockSpec((NC, BLK, 1), lambda i: (0, i, 0)),
            pl.BlockSpec((BLK, 1), lambda i: (i, 0)),
            pl.BlockSpec((1, C), lambda i: (0, 0)),
            pl.BlockSpec((1, C), lambda i: (0, 0)),
            pl.BlockSpec((1, C), lambda i: (0, 0)),
        ],
        out_specs=pl.BlockSpec((BLK, C), lambda i: (i, 0)),
        out_shape=jax.ShapeDtypeStruct((N, C), jnp.float32),
    )(s_parts, x, w1, w2, b)


def _tc_dense_body(a_ref, h_ref, w1_ref, w2_ref, b_ref, o_ref):
    agg = a_ref[0] + a_ref[1]
    hn = (jnp.dot(agg, w1_ref[...], preferred_element_type=jnp.float32)
          + jnp.dot(h_ref[...], w2_ref[...], preferred_element_type=jnp.float32)
          + b_ref[...])
    o_ref[...] = jnp.maximum(hn, 0.0)


def _tc_dense(a, h, w1, w2, b):
    return pl.pallas_call(
        _tc_dense_body,
        grid=(NGRID,),
        in_specs=[
            pl.BlockSpec((NC, BLK, C), lambda i: (0, i, 0)),
            pl.BlockSpec((BLK, C), lambda i: (i, 0)),
            pl.BlockSpec((C, C), lambda i: (0, 0)),
            pl.BlockSpec((C, C), lambda i: (0, 0)),
            pl.BlockSpec((1, C), lambda i: (0, 0)),
        ],
        out_specs=pl.BlockSpec((BLK, C), lambda i: (i, 0)),
        out_shape=jax.ShapeDtypeStruct((N, C), jnp.float32),
    )(a, h, w1, w2, b)


def _tc_final_body(a_ref, h_ref, w1_ref, w2_ref, b_ref, seg_ref, wd_ref,
                   bd_ref, o_ref, pool_acc, cnt_acc):
    i = pl.program_id(0)

    @pl.when(i == 0)
    def _init():
        pool_acc[...] = jnp.zeros_like(pool_acc)
        cnt_acc[...] = jnp.zeros_like(cnt_acc)

    agg = a_ref[0] + a_ref[1]
    h3 = (jnp.dot(agg, w1_ref[...], preferred_element_type=jnp.float32)
          + jnp.dot(h_ref[...], w2_ref[...], preferred_element_type=jnp.float32)
          + b_ref[...])
    h3 = jnp.maximum(h3, 0.0)

    onehot = (seg_ref[...] == lax.broadcasted_iota(jnp.int32, (BLK, B), 1))
    onehot = onehot.astype(jnp.float32)
    dn = (((0,), (0,)), ((), ()))
    pool_acc[...] += lax.dot_general(onehot, h3, dn,
                                     preferred_element_type=jnp.float32)
    cnt_acc[...] += lax.dot_general(onehot, jnp.ones((BLK, C), jnp.float32),
                                    dn, preferred_element_type=jnp.float32)

    @pl.when(i == NGRID - 1)
    def _finish():
        pooled = pool_acc[...] / jnp.maximum(cnt_acc[...], 1.0)
        z = jnp.dot(pooled, wd_ref[...],
                    preferred_element_type=jnp.float32) + bd_ref[...]
        o_ref[...] = jax.nn.sigmoid(z)


def _tc_final(a, h, w1, w2, b, seg, wd, bd):
    return pl.pallas_call(
        _tc_final_body,
        grid=(NGRID,),
        in_specs=[
            pl.BlockSpec((NC, BLK, C), lambda i: (0, i, 0)),
            pl.BlockSpec((BLK, C), lambda i: (i, 0)),
            pl.BlockSpec((C, C), lambda i: (0, 0)),
            pl.BlockSpec((C, C), lambda i: (0, 0)),
            pl.BlockSpec((1, C), lambda i: (0, 0)),
            pl.BlockSpec((BLK, 1), lambda i: (i, 0)),
            pl.BlockSpec((C, 1), lambda i: (0, 0)),
            pl.BlockSpec((1, 1), lambda i: (0, 0)),
        ],
        out_specs=pl.BlockSpec((B, 1), lambda i: (0, 0)),
        out_shape=jax.ShapeDtypeStruct((B, 1), jnp.float32),
        scratch_shapes=[
            pltpu.VMEM((B, C), jnp.float32),
            pltpu.VMEM((B, C), jnp.float32),
        ],
    )(a, h, w1, w2, b, seg, wd, bd)


def kernel(x, edge_index, edge_weight, segment_ids,
           W1_1, W2_1, b1, W1_2, W2_2, b2, W1_3, W2_3, b3, Wd, bd):
    src = edge_index[0].reshape(E // CH, CH)
    dst = edge_index[1].reshape(E // CH, CH)
    ew2 = edge_weight.reshape(E // CH, CH)
    x_flat = x.reshape(N)

    s_parts = _sc_scalar_agg(x_flat, src, dst, ew2)
    h1 = _tc_dense1(s_parts.reshape(NC, N, 1), x,
                    W1_1, W2_1, b1.reshape(1, C))

    a2 = _sc_row_agg(h1, src, dst, ew2)
    h2 = _tc_dense(a2, h1, W1_2, W2_2, b2.reshape(1, C))

    a3 = _sc_row_agg(h2, src, dst, ew2)
    out = _tc_final(a3, h2, W1_3, W2_3, b3.reshape(1, C),
                    segment_ids.reshape(N, 1), Wd, bd.reshape(1, 1))
    return out


# speedup vs baseline: 23.0785x; 1.1350x over previous
import functools

import jax
import jax.numpy as jnp
from jax import lax
from jax.experimental import pallas as pl
from jax.experimental.pallas import tpu as pltpu
from jax.experimental.pallas import tpu_sc as plsc

N = 50000
E = 1600000
B = 32
C = 32

NC = 2
NS = 16
NW = NC * NS
L = 16

TILE_E = E // NW
CH = 80
NSUB = 5
BE = CH * NSUB
NBLK = TILE_E // BE
NSUB1 = 25
BE1 = CH * NSUB1
NBLK1 = TILE_E // BE1

WR = 200

ACC1_PAD = 51200
Z1 = ACC1_PAD // NS
W1T = 10
WCH1 = N // W1T

BLK = 2000
NGRID = N // BLK

_mesh = plsc.VectorSubcoreMesh(core_axis_name="c", subcore_axis_name="s")


@functools.partial(
    pl.kernel,
    out_type=jax.ShapeDtypeStruct((NC * N,), jnp.float32),
    mesh=_mesh,
    scratch_types=[
        pltpu.VMEM((NSUB1, CH), jnp.int32),
        pltpu.VMEM((NSUB1, CH), jnp.int32),
        pltpu.VMEM((NSUB1, CH), jnp.float32),
        pltpu.VMEM((NSUB1, CH), jnp.float32),
        pltpu.VMEM((Z1,), jnp.float32),
        pltpu.VMEM((WCH1,), jnp.float32),
        pltpu.VMEM_SHARED((ACC1_PAD,), jnp.float32),
        pltpu.SemaphoreType.DMA,
        pltpu.SemaphoreType.DMA,
        pltpu.SemaphoreType.DMA,
    ],
    compiler_params=pltpu.CompilerParams(use_tc_tiling_on_sc=False),
)
def _sc_scalar_agg(x_hbm, src_hbm, dst_hbm, ew_hbm, out_hbm,
                   srcb, dstb, ewb, xb, zbuf, wbuf, acc,
                   isem, gsem, ssem):
    cidx = lax.axis_index("c")
    sidx = lax.axis_index("s")
    zeros = jnp.zeros((L,), jnp.float32)

    @pl.loop(0, Z1 // L)
    def _zero_stage(i):
        zbuf[pl.ds(i * L, L)] = zeros

    pltpu.sync_copy(zbuf, acc.at[pl.ds(sidx * Z1, Z1)])
    plsc.subcore_barrier()

    row0 = (cidx * NS + sidx) * (TILE_E // CH)

    @pl.loop(0, NBLK1)
    def _block(g):
        rbase = row0 + g * NSUB1
        c1 = pltpu.async_copy(src_hbm.at[pl.ds(rbase, NSUB1)], srcb, isem)
        c2 = pltpu.async_copy(dst_hbm.at[pl.ds(rbase, NSUB1)], dstb, isem)
        c3 = pltpu.async_copy(ew_hbm.at[pl.ds(rbase, NSUB1)], ewb, isem)
        c1.wait()
        c2.wait()
        c3.wait()
        gathers = [
            pltpu.async_copy(x_hbm.at[srcb.at[k]], xb.at[k], gsem)
            for k in range(NSUB1)
        ]
        for cp in gathers:
            cp.wait()

        @pl.loop(0, NSUB1)
        def _mul(k):
            for j in range(CH // L):
                sl = pl.ds(j * L, L)
                xb[k, sl] = xb[k, sl] * ewb[k, sl]

        scatters = [
            pltpu.async_copy(xb.at[k], acc.at[dstb.at[k]], ssem, add=True)
            for k in range(NSUB1)
        ]
        for cp in scatters:
            cp.wait()

    plsc.subcore_barrier()

    @pl.when(sidx < W1T)
    def _writeout():
        pltpu.sync_copy(acc.at[pl.ds(sidx * WCH1, WCH1)], wbuf)
        pltpu.sync_copy(wbuf, out_hbm.at[pl.ds(cidx * N + sidx * WCH1, WCH1)])


@functools.partial(
    pl.kernel,
    out_type=jax.ShapeDtypeStruct((NC, N, C), jnp.float32),
    mesh=_mesh,
    scratch_types=[
        pltpu.VMEM((NSUB, CH), jnp.int32),
        pltpu.VMEM((NSUB, CH), jnp.int32),
        pltpu.VMEM((NSUB, CH), jnp.float32),
        pltpu.VMEM((BE, C), jnp.float32),
        pltpu.VMEM((WR, C), jnp.float32),
        pltpu.VMEM_SHARED((N, C), jnp.float32),
        pltpu.SemaphoreType.DMA,
        [pltpu.SemaphoreType.DMA] * NSUB,
        pltpu.SemaphoreType.DMA,
    ],
    compiler_params=pltpu.CompilerParams(use_tc_tiling_on_sc=False),
)
def _sc_row_agg(h_hbm, src_hbm, dst_hbm, ew_hbm, out_hbm,
                srcb, dstb, ewb, rowsb, wbuf, acc, isem, gsems, ssem):
    cidx = lax.axis_index("c")
    sidx = lax.axis_index("s")
    zeros = jnp.zeros((L,), jnp.float32)
    iota = lax.iota(jnp.int32, L)

    @pl.loop(0, WR)
    def _zero_stage(r):
        wbuf[r, pl.ds(0, L)] = zeros
        wbuf[r, pl.ds(L, L)] = zeros

    @pl.when(sidx < W1T)
    def _zero_acc():
        @pl.loop(0, WCH1 // WR)
        def _(i):
            pltpu.sync_copy(wbuf, acc.at[pl.ds(sidx * WCH1 + i * WR, WR)])

    plsc.subcore_barrier()

    row0 = (cidx * NS + sidx) * (TILE_E // CH)
    gdn = lax.GatherDimensionNumbers(
        offset_dims=(), collapsed_slice_dims=(0,), start_index_map=(0,))

    @pl.loop(0, NBLK)
    def _block(g):
        rbase = row0 + g * NSUB
        c1 = pltpu.async_copy(src_hbm.at[pl.ds(rbase, NSUB)], srcb, isem)
        c2 = pltpu.async_copy(dst_hbm.at[pl.ds(rbase, NSUB)], dstb, isem)
        c3 = pltpu.async_copy(ew_hbm.at[pl.ds(rbase, NSUB)], ewb, isem)
        c1.wait()
        c2.wait()
        c3.wait()
        gathers = [
            pltpu.async_copy(h_hbm.at[srcb.at[k]],
                             rowsb.at[pl.ds(k * CH, CH)], gsems[k])
            for k in range(NSUB)
        ]
        scatters = []
        for k in range(NSUB):
            gathers[k].wait()

            @pl.loop(0, CH // L)
            def _scale(jj, k=k):
                wv = ewb[k, pl.ds(jj * L, L)]
                for el in range(L):
                    w = lax.gather(wv, jnp.full((L, 1), el, jnp.int32), gdn,
                                   (1,),
                                   mode=lax.GatherScatterMode.PROMISE_IN_BOUNDS)
                    e = k * CH + jj * L + el
                    rowsb[e, pl.ds(0, L)] = rowsb[e, pl.ds(0, L)] * w
                    rowsb[e, pl.ds(L, L)] = rowsb[e, pl.ds(L, L)] * w
            scatters.append(
                pltpu.async_copy(rowsb.at[pl.ds(k * CH, CH)],
                                 acc.at[dstb.at[k]], ssem, add=True))
        for cp in scatters:
            cp.wait()

    plsc.subcore_barrier()

    @pl.when(sidx < W1T)
    def _writeout():
        @pl.loop(0, WCH1 // WR)
        def _(i):
            r0 = sidx * WCH1 + i * WR
            pltpu.sync_copy(acc.at[pl.ds(r0, WR)], wbuf)
            pltpu.sync_copy(wbuf, out_hbm.at[cidx, pl.ds(r0, WR)])


def _tc_dense1_body(sp_ref, x_ref, w1_ref, w2_ref, b_ref, o_ref):
    s = sp_ref[0] + sp_ref[1]
    h = s * w1_ref[...] + x_ref[...] * w2_ref[...] + b_ref[...]
    o_ref[...] = jnp.maximum(h, 0.0)


def _tc_dense1(s_parts, x, w1, w2, b):
    return pl.pallas_call(
        _tc_dense1_body,
        grid=(NGRID,),
        in_specs=[
            pl.BlockSpec((NC, BLK, 1), lambda i: (0, i, 0)),
            pl.BlockSpec((BLK, 1), lambda i: (i, 0)),
            pl.BlockSpec((1, C), lambda i: (0, 0)),
            pl.BlockSpec((1, C), lambda i: (0, 0)),
            pl.BlockSpec((1, C), lambda i: (0, 0)),
        ],
        out_specs=pl.BlockSpec((BLK, C), lambda i: (i, 0)),
        out_shape=jax.ShapeDtypeStruct((N, C), jnp.float32),
    )(s_parts, x, w1, w2, b)


def _tc_dense_body(a_ref, h_ref, w1_ref, w2_ref, b_ref, o_ref):
    agg = a_ref[0] + a_ref[1]
    hn = (jnp.dot(agg, w1_ref[...], preferred_element_type=jnp.float32)
          + jnp.dot(h_ref[...], w2_ref[...], preferred_element_type=jnp.float32)
          + b_ref[...])
    o_ref[...] = jnp.maximum(hn, 0.0)


def _tc_dense(a, h, w1, w2, b):
    return pl.pallas_call(
        _tc_dense_body,
        grid=(NGRID,),
        in_specs=[
            pl.BlockSpec((NC, BLK, C), lambda i: (0, i, 0)),
            pl.BlockSpec((BLK, C), lambda i: (i, 0)),
            pl.BlockSpec((C, C), lambda i: (0, 0)),
            pl.BlockSpec((C, C), lambda i: (0, 0)),
            pl.BlockSpec((1, C), lambda i: (0, 0)),
        ],
        out_specs=pl.BlockSpec((BLK, C), lambda i: (i, 0)),
        out_shape=jax.ShapeDtypeStruct((N, C), jnp.float32),
    )(a, h, w1, w2, b)


def _tc_final_body(a_ref, h_ref, w1_ref, w2_ref, b_ref, seg_ref, wd_ref,
                   bd_ref, o_ref, pool_acc, cnt_acc):
    i = pl.program_id(0)

    @pl.when(i == 0)
    def _init():
        pool_acc[...] = jnp.zeros_like(pool_acc)
        cnt_acc[...] = jnp.zeros_like(cnt_acc)

    agg = a_ref[0] + a_ref[1]
    h3 = (jnp.dot(agg, w1_ref[...], preferred_element_type=jnp.float32)
          + jnp.dot(h_ref[...], w2_ref[...], preferred_element_type=jnp.float32)
          + b_ref[...])
    h3 = jnp.maximum(h3, 0.0)

    onehot = (seg_ref[...] == lax.broadcasted_iota(jnp.int32, (BLK, B), 1))
    onehot = onehot.astype(jnp.float32)
    dn = (((0,), (0,)), ((), ()))
    pool_acc[...] += lax.dot_general(onehot, h3, dn,
                                     preferred_element_type=jnp.float32)
    cnt_acc[...] += lax.dot_general(onehot, jnp.ones((BLK, C), jnp.float32),
                                    dn, preferred_element_type=jnp.float32)

    @pl.when(i == NGRID - 1)
    def _finish():
        pooled = pool_acc[...] / jnp.maximum(cnt_acc[...], 1.0)
        z = jnp.dot(pooled, wd_ref[...],
                    preferred_element_type=jnp.float32) + bd_ref[...]
        o_ref[...] = jax.nn.sigmoid(z)


def _tc_final(a, h, w1, w2, b, seg, wd, bd):
    return pl.pallas_call(
        _tc_final_body,
        grid=(NGRID,),
        in_specs=[
            pl.BlockSpec((NC, BLK, C), lambda i: (0, i, 0)),
            pl.BlockSpec((BLK, C), lambda i: (i, 0)),
            pl.BlockSpec((C, C), lambda i: (0, 0)),
            pl.BlockSpec((C, C), lambda i: (0, 0)),
            pl.BlockSpec((1, C), lambda i: (0, 0)),
            pl.BlockSpec((BLK, 1), lambda i: (i, 0)),
            pl.BlockSpec((C, 1), lambda i: (0, 0)),
            pl.BlockSpec((1, 1), lambda i: (0, 0)),
        ],
        out_specs=pl.BlockSpec((B, 1), lambda i: (0, 0)),
        out_shape=jax.ShapeDtypeStruct((B, 1), jnp.float32),
        scratch_shapes=[
            pltpu.VMEM((B, C), jnp.float32),
            pltpu.VMEM((B, C), jnp.float32),
        ],
    )(a, h, w1, w2, b, seg, wd, bd)


def kernel(x, edge_index, edge_weight, segment_ids,
           W1_1, W2_1, b1, W1_2, W2_2, b2, W1_3, W2_3, b3, Wd, bd):
    src = edge_index[0].reshape(E // CH, CH)
    dst = edge_index[1].reshape(E // CH, CH)
    ew2 = edge_weight.reshape(E // CH, CH)
    x_flat = x.reshape(N)

    s_parts = _sc_scalar_agg(x_flat, src, dst, ew2)
    h1 = _tc_dense1(s_parts.reshape(NC, N, 1), x,
                    W1_1, W2_1, b1.reshape(1, C))

    a2 = _sc_row_agg(h1, src, dst, ew2)
    h2 = _tc_dense(a2, h1, W1_2, W2_2, b2.reshape(1, C))

    a3 = _sc_row_agg(h2, src, dst, ew2)
    out = _tc_final(a3, h2, W1_3, W2_3, b3.reshape(1, C),
                    segment_ids.reshape(N, 1), Wd, bd.reshape(1, 1))
    return out


# speedup vs baseline: 26.8285x; 1.1625x over previous
import functools

import jax
import jax.numpy as jnp
from jax import lax
from jax.experimental import pallas as pl
from jax.experimental.pallas import tpu as pltpu
from jax.experimental.pallas import tpu_sc as plsc

N = 50000
E = 1600000
B = 32
C = 32

NC = 2
NS = 16
NW = NC * NS
L = 16

TILE_E = E // NW
CH = 80
NSUB = 5
BE = CH * NSUB
NBLK = TILE_E // BE
NSUB1 = 25
BE1 = CH * NSUB1
NBLK1 = TILE_E // BE1

WR = 200

ACC1_PAD = 51200
Z1 = ACC1_PAD // NS
W1T = 10
WCH1 = N // W1T

NP = N // 4
BLK = 500
NGRID = NP // BLK

_mesh = plsc.VectorSubcoreMesh(core_axis_name="c", subcore_axis_name="s")


@functools.partial(
    pl.kernel,
    out_type=jax.ShapeDtypeStruct((NC * N,), jnp.float32),
    mesh=_mesh,
    scratch_types=[
        pltpu.VMEM((NSUB1, CH), jnp.int32),
        pltpu.VMEM((NSUB1, CH), jnp.int32),
        pltpu.VMEM((NSUB1, CH), jnp.float32),
        pltpu.VMEM((NSUB1, CH), jnp.float32),
        pltpu.VMEM((Z1,), jnp.float32),
        pltpu.VMEM((WCH1,), jnp.float32),
        pltpu.VMEM_SHARED((ACC1_PAD,), jnp.float32),
        pltpu.SemaphoreType.DMA,
        pltpu.SemaphoreType.DMA,
        pltpu.SemaphoreType.DMA,
    ],
    compiler_params=pltpu.CompilerParams(use_tc_tiling_on_sc=False),
)
def _sc_scalar_agg(x_hbm, src_hbm, dst_hbm, ew_hbm, out_hbm,
                   srcb, dstb, ewb, xb, zbuf, wbuf, acc,
                   isem, gsem, ssem):
    cidx = lax.axis_index("c")
    sidx = lax.axis_index("s")
    zeros = jnp.zeros((L,), jnp.float32)

    @pl.loop(0, Z1 // L)
    def _zero_stage(i):
        zbuf[pl.ds(i * L, L)] = zeros

    pltpu.sync_copy(zbuf, acc.at[pl.ds(sidx * Z1, Z1)])
    plsc.subcore_barrier()

    row0 = (cidx * NS + sidx) * (TILE_E // CH)

    @pl.loop(0, NBLK1)
    def _block(g):
        rbase = row0 + g * NSUB1
        c1 = pltpu.async_copy(src_hbm.at[pl.ds(rbase, NSUB1)], srcb, isem)
        c2 = pltpu.async_copy(dst_hbm.at[pl.ds(rbase, NSUB1)], dstb, isem)
        c3 = pltpu.async_copy(ew_hbm.at[pl.ds(rbase, NSUB1)], ewb, isem)
        c1.wait()
        c2.wait()
        c3.wait()
        gathers = [
            pltpu.async_copy(x_hbm.at[srcb.at[k]], xb.at[k], gsem)
            for k in range(NSUB1)
        ]
        for cp in gathers:
            cp.wait()

        @pl.loop(0, NSUB1)
        def _mul(k):
            for j in range(CH // L):
                sl = pl.ds(j * L, L)
                xb[k, sl] = xb[k, sl] * ewb[k, sl]

        scatters = [
            pltpu.async_copy(xb.at[k], acc.at[dstb.at[k]], ssem, add=True)
            for k in range(NSUB1)
        ]
        for cp in scatters:
            cp.wait()

    plsc.subcore_barrier()

    @pl.when(sidx < W1T)
    def _writeout():
        pltpu.sync_copy(acc.at[pl.ds(sidx * WCH1, WCH1)], wbuf)
        pltpu.sync_copy(wbuf, out_hbm.at[pl.ds(cidx * N + sidx * WCH1, WCH1)])


@functools.partial(
    pl.kernel,
    out_type=jax.ShapeDtypeStruct((NC, N, C), jnp.float32),
    mesh=_mesh,
    scratch_types=[
        pltpu.VMEM((NSUB, CH), jnp.int32),
        pltpu.VMEM((NSUB, CH), jnp.int32),
        pltpu.VMEM((NSUB, CH), jnp.float32),
        pltpu.VMEM((BE, C), jnp.float32),
        pltpu.VMEM((WR, C), jnp.float32),
        pltpu.VMEM_SHARED((N, C), jnp.float32),
        pltpu.SemaphoreType.DMA,
        [pltpu.SemaphoreType.DMA] * NSUB,
        pltpu.SemaphoreType.DMA,
    ],
    compiler_params=pltpu.CompilerParams(use_tc_tiling_on_sc=False),
)
def _sc_row_agg(h_hbm, src_hbm, dst_hbm, ew_hbm, out_hbm,
                srcb, dstb, ewb, rowsb, wbuf, acc, isem, gsems, ssem):
    cidx = lax.axis_index("c")
    sidx = lax.axis_index("s")
    zeros = jnp.zeros((L,), jnp.float32)
    iota = lax.iota(jnp.int32, L)

    @pl.loop(0, WR)
    def _zero_stage(r):
        wbuf[r, pl.ds(0, L)] = zeros
        wbuf[r, pl.ds(L, L)] = zeros

    @pl.when(sidx < W1T)
    def _zero_acc():
        @pl.loop(0, WCH1 // WR)
        def _(i):
            pltpu.sync_copy(wbuf, acc.at[pl.ds(sidx * WCH1 + i * WR, WR)])

    plsc.subcore_barrier()

    row0 = (cidx * NS + sidx) * (TILE_E // CH)
    gdn = lax.GatherDimensionNumbers(
        offset_dims=(), collapsed_slice_dims=(0,), start_index_map=(0,))

    @pl.loop(0, NBLK)
    def _block(g):
        rbase = row0 + g * NSUB
        c1 = pltpu.async_copy(src_hbm.at[pl.ds(rbase, NSUB)], srcb, isem)
        c2 = pltpu.async_copy(dst_hbm.at[pl.ds(rbase, NSUB)], dstb, isem)
        c3 = pltpu.async_copy(ew_hbm.at[pl.ds(rbase, NSUB)], ewb, isem)
        c1.wait()
        c2.wait()
        c3.wait()
        gathers = [
            pltpu.async_copy(h_hbm.at[srcb.at[k]],
                             rowsb.at[pl.ds(k * CH, CH)], gsems[k])
            for k in range(NSUB)
        ]
        scatters = []
        for k in range(NSUB):
            gathers[k].wait()

            @pl.loop(0, CH // L)
            def _scale(jj, k=k):
                wv = ewb[k, pl.ds(jj * L, L)]
                for el in range(L):
                    w = lax.gather(wv, jnp.full((L, 1), el, jnp.int32), gdn,
                                   (1,),
                                   mode=lax.GatherScatterMode.PROMISE_IN_BOUNDS)
                    e = k * CH + jj * L + el
                    rowsb[e, pl.ds(0, L)] = rowsb[e, pl.ds(0, L)] * w
                    rowsb[e, pl.ds(L, L)] = rowsb[e, pl.ds(L, L)] * w
            scatters.append(
                pltpu.async_copy(rowsb.at[pl.ds(k * CH, CH)],
                                 acc.at[dstb.at[k]], ssem, add=True))
        for cp in scatters:
            cp.wait()

    plsc.subcore_barrier()

    @pl.when(sidx < W1T)
    def _writeout():
        @pl.loop(0, WCH1 // WR)
        def _(i):
            r0 = sidx * WCH1 + i * WR
            pltpu.sync_copy(acc.at[pl.ds(r0, WR)], wbuf)
            pltpu.sync_copy(wbuf, out_hbm.at[cidx, pl.ds(r0, WR)])


def _tc_dense1_body(sp_ref, x_ref, pw1_ref, pw2_ref, b_ref, o_ref):
    s = sp_ref[0, 0] + sp_ref[1, 0]
    h = (jnp.dot(s, pw1_ref[...], preferred_element_type=jnp.float32)
         + jnp.dot(x_ref[0], pw2_ref[...],
                   preferred_element_type=jnp.float32)
         + b_ref[...])
    o_ref[0] = jnp.maximum(h, 0.0)


def _tc_dense1(s_parts, x4, pw1, pw2, btile):
    return pl.pallas_call(
        _tc_dense1_body,
        grid=(NGRID,),
        in_specs=[
            pl.BlockSpec((NC, 1, BLK, 4), lambda i: (0, i, 0, 0)),
            pl.BlockSpec((1, BLK, 4), lambda i: (i, 0, 0)),
            pl.BlockSpec((4, 4 * C), lambda i: (0, 0)),
            pl.BlockSpec((4, 4 * C), lambda i: (0, 0)),
            pl.BlockSpec((1, 4 * C), lambda i: (0, 0)),
        ],
        out_specs=pl.BlockSpec((1, BLK, 4 * C), lambda i: (i, 0, 0)),
        out_shape=jax.ShapeDtypeStruct((NGRID, BLK, 4 * C), jnp.float32),
    )(s_parts, x4, pw1, pw2, btile)


def _tc_dense_body(a_ref, h_ref, w1_ref, w2_ref, b_ref, o_ref):
    agg = a_ref[0, 0] + a_ref[1, 0]
    hn = (jnp.dot(agg, w1_ref[...], preferred_element_type=jnp.float32)
          + jnp.dot(h_ref[0], w2_ref[...], preferred_element_type=jnp.float32)
          + b_ref[...])
    o_ref[0] = jnp.maximum(hn, 0.0)


def _tc_dense(a, h, w1blk, w2blk, btile):
    return pl.pallas_call(
        _tc_dense_body,
        grid=(NGRID,),
        in_specs=[
            pl.BlockSpec((NC, 1, BLK, 4 * C), lambda i: (0, i, 0, 0)),
            pl.BlockSpec((1, BLK, 4 * C), lambda i: (i, 0, 0)),
            pl.BlockSpec((4 * C, 4 * C), lambda i: (0, 0)),
            pl.BlockSpec((4 * C, 4 * C), lambda i: (0, 0)),
            pl.BlockSpec((1, 4 * C), lambda i: (0, 0)),
        ],
        out_specs=pl.BlockSpec((1, BLK, 4 * C), lambda i: (i, 0, 0)),
        out_shape=jax.ShapeDtypeStruct((NGRID, BLK, 4 * C), jnp.float32),
    )(a, h, w1blk, w2blk, btile)


def _tc_final_body(a_ref, h_ref, w1_ref, w2_ref, b_ref, seg_ref, wd_ref,
                   bd_ref, o_ref, pool_acc, cnt_acc):
    i = pl.program_id(0)

    @pl.when(i == 0)
    def _init():
        pool_acc[...] = jnp.zeros_like(pool_acc)
        cnt_acc[...] = jnp.zeros_like(cnt_acc)

    agg = a_ref[0, 0] + a_ref[1, 0]
    h3 = (jnp.dot(agg, w1_ref[...], preferred_element_type=jnp.float32)
          + jnp.dot(h_ref[0], w2_ref[...], preferred_element_type=jnp.float32)
          + b_ref[...])
    h3 = jnp.maximum(h3, 0.0)

    dn = (((0,), (0,)), ((), ()))
    for u in range(4):
        onehot = (seg_ref[0][:, u:u + 1]
                  == lax.broadcasted_iota(jnp.int32, (BLK, B), 1))
        onehot = onehot.astype(jnp.float32)
        rows_u = h3[:, u * C:(u + 1) * C]
        pool_acc[...] += lax.dot_general(onehot, rows_u, dn,
                                         preferred_element_type=jnp.float32)
        cnt_acc[...] += lax.dot_general(
            onehot, jnp.ones((BLK, C), jnp.float32), dn,
            preferred_element_type=jnp.float32)

    @pl.when(i == NGRID - 1)
    def _finish():
        pooled = pool_acc[...] / jnp.maximum(cnt_acc[...], 1.0)
        z = jnp.dot(pooled, wd_ref[...],
                    preferred_element_type=jnp.float32) + bd_ref[...]
        o_ref[...] = jax.nn.sigmoid(z)


def _tc_final(a, h, w1blk, w2blk, btile, seg4, wd, bd):
    return pl.pallas_call(
        _tc_final_body,
        grid=(NGRID,),
        in_specs=[
            pl.BlockSpec((NC, 1, BLK, 4 * C), lambda i: (0, i, 0, 0)),
            pl.BlockSpec((1, BLK, 4 * C), lambda i: (i, 0, 0)),
            pl.BlockSpec((4 * C, 4 * C), lambda i: (0, 0)),
            pl.BlockSpec((4 * C, 4 * C), lambda i: (0, 0)),
            pl.BlockSpec((1, 4 * C), lambda i: (0, 0)),
            pl.BlockSpec((1, BLK, 4), lambda i: (i, 0, 0)),
            pl.BlockSpec((C, 1), lambda i: (0, 0)),
            pl.BlockSpec((1, 1), lambda i: (0, 0)),
        ],
        out_specs=pl.BlockSpec((B, 1), lambda i: (0, 0)),
        out_shape=jax.ShapeDtypeStruct((B, 1), jnp.float32),
        scratch_shapes=[
            pltpu.VMEM((B, C), jnp.float32),
            pltpu.VMEM((B, C), jnp.float32),
        ],
    )(a, h, w1blk, w2blk, btile, seg4, wd, bd)


def kernel(x, edge_index, edge_weight, segment_ids,
           W1_1, W2_1, b1, W1_2, W2_2, b2, W1_3, W2_3, b3, Wd, bd):
    src = edge_index[0].reshape(E // CH, CH)
    dst = edge_index[1].reshape(E // CH, CH)
    ew2 = edge_weight.reshape(E // CH, CH)
    x_flat = x.reshape(N)

    eye4 = jnp.eye(4, dtype=jnp.float32)
    pw1 = jnp.kron(eye4, W1_1)
    pw2 = jnp.kron(eye4, W2_1)
    w1blk2 = jnp.kron(eye4, W1_2)
    w2blk2 = jnp.kron(eye4, W2_2)
    w1blk3 = jnp.kron(eye4, W1_3)
    w2blk3 = jnp.kron(eye4, W2_3)

    s_parts = _sc_scalar_agg(x_flat, src, dst, ew2)
    h1p = _tc_dense1(s_parts.reshape(NC, NGRID, BLK, 4),
                     x.reshape(NGRID, BLK, 4),
                     pw1, pw2, jnp.tile(b1, 4).reshape(1, 4 * C))

    a2 = _sc_row_agg(h1p.reshape(N, C), src, dst, ew2)
    h2p = _tc_dense(a2.reshape(NC, NGRID, BLK, 4 * C), h1p, w1blk2, w2blk2,
                    jnp.tile(b2, 4).reshape(1, 4 * C))

    a3 = _sc_row_agg(h2p.reshape(N, C), src, dst, ew2)
    out = _tc_final(a3.reshape(NC, NGRID, BLK, 4 * C), h2p, w1blk3, w2blk3,
                    jnp.tile(b3, 4).reshape(1, 4 * C),
                    segment_ids.reshape(NGRID, BLK, 4), Wd, bd.reshape(1, 1))
    return out


# speedup vs baseline: 31.3723x; 1.1694x over previous
import functools

import jax
import jax.numpy as jnp
from jax import lax
from jax.experimental import pallas as pl
from jax.experimental.pallas import tpu as pltpu
from jax.experimental.pallas import tpu_sc as plsc

N = 50000
E = 1600000
B = 32
C = 32

NC = 2
NS = 16
NW = NC * NS
L = 16

TILE_E = E // NW
CH = 80
NSUB = 5
BE = CH * NSUB
NBLK = TILE_E // BE
NSUB1 = 25
BE1 = CH * NSUB1
NBLK1 = TILE_E // BE1

WR = 200

ACC1_PAD = 51200
Z1 = ACC1_PAD // NS
W1T = 10
WCH1 = N // W1T

NP = N // 4
BLK = 500
NGRID = NP // BLK

_mesh = plsc.VectorSubcoreMesh(core_axis_name="c", subcore_axis_name="s")


@functools.partial(
    pl.kernel,
    out_type=jax.ShapeDtypeStruct((NC * N,), jnp.float32),
    mesh=_mesh,
    scratch_types=[
        pltpu.VMEM((NSUB1, CH), jnp.int32),
        pltpu.VMEM((NSUB1, CH), jnp.int32),
        pltpu.VMEM((NSUB1, CH), jnp.float32),
        pltpu.VMEM((NSUB1, CH), jnp.float32),
        pltpu.VMEM((Z1,), jnp.float32),
        pltpu.VMEM((WCH1,), jnp.float32),
        pltpu.VMEM_SHARED((ACC1_PAD,), jnp.float32),
        pltpu.SemaphoreType.DMA,
        pltpu.SemaphoreType.DMA,
        pltpu.SemaphoreType.DMA,
    ],
    compiler_params=pltpu.CompilerParams(use_tc_tiling_on_sc=False),
)
def _sc_scalar_agg(x_hbm, src_hbm, dst_hbm, ew_hbm, out_hbm,
                   srcb, dstb, ewb, xb, zbuf, wbuf, acc,
                   isem, gsem, ssem):
    cidx = lax.axis_index("c")
    sidx = lax.axis_index("s")
    zeros = jnp.zeros((L,), jnp.float32)

    @pl.loop(0, Z1 // L)
    def _zero_stage(i):
        zbuf[pl.ds(i * L, L)] = zeros

    pltpu.sync_copy(zbuf, acc.at[pl.ds(sidx * Z1, Z1)])
    plsc.subcore_barrier()

    row0 = (cidx * NS + sidx) * (TILE_E // CH)

    @pl.loop(0, NBLK1)
    def _block(g):
        rbase = row0 + g * NSUB1
        c1 = pltpu.async_copy(src_hbm.at[pl.ds(rbase, NSUB1)], srcb, isem)
        c2 = pltpu.async_copy(dst_hbm.at[pl.ds(rbase, NSUB1)], dstb, isem)
        c3 = pltpu.async_copy(ew_hbm.at[pl.ds(rbase, NSUB1)], ewb, isem)
        c1.wait()
        c2.wait()
        c3.wait()
        gathers = [
            pltpu.async_copy(x_hbm.at[srcb.at[k]], xb.at[k], gsem)
            for k in range(NSUB1)
        ]
        for cp in gathers:
            cp.wait()

        @pl.loop(0, NSUB1)
        def _mul(k):
            for j in range(CH // L):
                sl = pl.ds(j * L, L)
                xb[k, sl] = xb[k, sl] * ewb[k, sl]

        scatters = [
            pltpu.async_copy(xb.at[k], acc.at[dstb.at[k]], ssem, add=True)
            for k in range(NSUB1)
        ]
        for cp in scatters:
            cp.wait()

    plsc.subcore_barrier()

    @pl.when(sidx < W1T)
    def _writeout():
        pltpu.sync_copy(acc.at[pl.ds(sidx * WCH1, WCH1)], wbuf)
        pltpu.sync_copy(wbuf, out_hbm.at[pl.ds(cidx * N + sidx * WCH1, WCH1)])


@functools.partial(
    pl.kernel,
    out_type=jax.ShapeDtypeStruct((NC, N, C), jnp.float32),
    mesh=_mesh,
    scratch_types=[
        pltpu.VMEM((NSUB, CH), jnp.int32),
        pltpu.VMEM((NSUB, CH), jnp.int32),
        pltpu.VMEM((NSUB, CH), jnp.float32),
        pltpu.VMEM((NSUB, CH), jnp.int32),
        pltpu.VMEM((NSUB, CH), jnp.int32),
        pltpu.VMEM((NSUB, CH), jnp.float32),
        pltpu.VMEM((BE, C), jnp.float32),
        pltpu.VMEM((BE, C), jnp.float32),
        pltpu.VMEM_SHARED((N, C), jnp.float32),
        pltpu.SemaphoreType.DMA,
        pltpu.SemaphoreType.DMA,
        [pltpu.SemaphoreType.DMA] * NSUB,
        [pltpu.SemaphoreType.DMA] * NSUB,
        pltpu.SemaphoreType.DMA,
        pltpu.SemaphoreType.DMA,
    ],
    compiler_params=pltpu.CompilerParams(use_tc_tiling_on_sc=False),
)
def _sc_row_agg(h_hbm, src_hbm, dst_hbm, ew_hbm, out_hbm,
                srcbA, dstbA, ewbA, srcbB, dstbB, ewbB, rowsA, rowsB, acc,
                isemA, isemB, gsemsA, gsemsB, ssemA, ssemB):
    cidx = lax.axis_index("c")
    sidx = lax.axis_index("s")
    zeros = jnp.zeros((L,), jnp.float32)
    gdn = lax.GatherDimensionNumbers(
        offset_dims=(), collapsed_slice_dims=(0,), start_index_map=(0,))

    @pl.loop(0, WR)
    def _zero_stage(r):
        rowsA[r, pl.ds(0, L)] = zeros
        rowsA[r, pl.ds(L, L)] = zeros

    @pl.when(sidx < W1T)
    def _zero_acc():
        @pl.loop(0, WCH1 // WR)
        def _(i):
            pltpu.sync_copy(rowsA.at[pl.ds(0, WR)],
                            acc.at[pl.ds(sidx * WCH1 + i * WR, WR)])

    plsc.subcore_barrier()

    row0 = (cidx * NS + sidx) * (TILE_E // CH)

    def fire_idx(rbase, sb, db, eb, sem):
        return [pltpu.async_copy(src_hbm.at[pl.ds(rbase, NSUB)], sb, sem),
                pltpu.async_copy(dst_hbm.at[pl.ds(rbase, NSUB)], db, sem),
                pltpu.async_copy(ew_hbm.at[pl.ds(rbase, NSUB)], eb, sem)]

    def drain_idx(rbase, sb, db, eb, sem):
        pltpu.make_async_copy(src_hbm.at[pl.ds(rbase, NSUB)], sb, sem).wait()
        pltpu.make_async_copy(dst_hbm.at[pl.ds(rbase, NSUB)], db, sem).wait()
        pltpu.make_async_copy(ew_hbm.at[pl.ds(rbase, NSUB)], eb, sem).wait()

    def fire_gathers(sb, rows, gsems):
        return [pltpu.async_copy(h_hbm.at[sb.at[k]],
                                 rows.at[pl.ds(k * CH, CH)], gsems[k])
                for k in range(NSUB)]

    def mult_scatter(k, eb, rows, db, ssem):
        @pl.loop(0, CH // L)
        def _scale(jj):
            wv = eb[k, pl.ds(jj * L, L)]
            for el in range(L):
                w = lax.gather(wv, jnp.full((L, 1), el, jnp.int32), gdn, (1,),
                               mode=lax.GatherScatterMode.PROMISE_IN_BOUNDS)
                e = k * CH + jj * L + el
                rows[e, pl.ds(0, L)] = rows[e, pl.ds(0, L)] * w
                rows[e, pl.ds(L, L)] = rows[e, pl.ds(L, L)] * w
        return pltpu.async_copy(rows.at[pl.ds(k * CH, CH)],
                                acc.at[db.at[k]], ssem, add=True)

    fire_idx(row0, srcbA, dstbA, ewbA, isemA)

    @pl.loop(0, (NBLK - 1) // 2)
    def _pipe(t):
        rbA = row0 + (2 * t) * NSUB
        rbB = rbA + NSUB
        drain_idx(rbA, srcbA, dstbA, ewbA, isemA)
        gA = fire_gathers(srcbA, rowsA, gsemsA)
        idxB = fire_idx(rbB, srcbB, dstbB, ewbB, isemB)
        scatA = []
        for k in (0, 1):
            gA[k].wait()
            scatA.append(mult_scatter(k, ewbA, rowsA, dstbA, ssemA))
        for cp in idxB:
            cp.wait()
        gB = fire_gathers(srcbB, rowsB, gsemsB)
        for k in (2, 3, 4):
            gA[k].wait()
            scatA.append(mult_scatter(k, ewbA, rowsA, dstbA, ssemA))
        for cp in scatA:
            cp.wait()
        scatB = []
        for k in range(NSUB):
            gB[k].wait()
            scatB.append(mult_scatter(k, ewbB, rowsB, dstbB, ssemB))
        fire_idx(rbA + 2 * NSUB, srcbA, dstbA, ewbA, isemA)
        for cp in scatB:
            cp.wait()

    rbT = row0 + (NBLK - 1) * NSUB
    drain_idx(rbT, srcbA, dstbA, ewbA, isemA)
    gT = fire_gathers(srcbA, rowsA, gsemsA)
    scatT = []
    for k in range(NSUB):
        gT[k].wait()
        scatT.append(mult_scatter(k, ewbA, rowsA, dstbA, ssemA))
    for cp in scatT:
        cp.wait()

    plsc.subcore_barrier()

    @pl.when(sidx < W1T)
    def _writeout():
        @pl.loop(0, WCH1 // WR)
        def _(i):
            r0 = sidx * WCH1 + i * WR
            pltpu.sync_copy(acc.at[pl.ds(r0, WR)], rowsA.at[pl.ds(0, WR)])
            pltpu.sync_copy(rowsA.at[pl.ds(0, WR)], out_hbm.at[cidx, pl.ds(r0, WR)])


def _tc_dense1_body(sp_ref, x_ref, pw1_ref, pw2_ref, b_ref, o_ref):
    s = sp_ref[0, 0] + sp_ref[1, 0]
    h = (jnp.dot(s, pw1_ref[...], preferred_element_type=jnp.float32)
         + jnp.dot(x_ref[0], pw2_ref[...],
                   preferred_element_type=jnp.float32)
         + b_ref[...])
    o_ref[0] = jnp.maximum(h, 0.0)


def _tc_dense1(s_parts, x4, pw1, pw2, btile):
    return pl.pallas_call(
        _tc_dense1_body,
        grid=(NGRID,),
        in_specs=[
            pl.BlockSpec((NC, 1, BLK, 4), lambda i: (0, i, 0, 0)),
            pl.BlockSpec((1, BLK, 4), lambda i: (i, 0, 0)),
            pl.BlockSpec((4, 4 * C), lambda i: (0, 0)),
            pl.BlockSpec((4, 4 * C), lambda i: (0, 0)),
            pl.BlockSpec((1, 4 * C), lambda i: (0, 0)),
        ],
        out_specs=pl.BlockSpec((1, BLK, 4 * C), lambda i: (i, 0, 0)),
        out_shape=jax.ShapeDtypeStruct((NGRID, BLK, 4 * C), jnp.float32),
    )(s_parts, x4, pw1, pw2, btile)


def _tc_dense_body(a_ref, h_ref, w1_ref, w2_ref, b_ref, o_ref):
    agg = a_ref[0, 0] + a_ref[1, 0]
    hn = (jnp.dot(agg, w1_ref[...], preferred_element_type=jnp.float32)
          + jnp.dot(h_ref[0], w2_ref[...], preferred_element_type=jnp.float32)
          + b_ref[...])
    o_ref[0] = jnp.maximum(hn, 0.0)


def _tc_dense(a, h, w1blk, w2blk, btile):
    return pl.pallas_call(
        _tc_dense_body,
        grid=(NGRID,),
        in_specs=[
            pl.BlockSpec((NC, 1, BLK, 4 * C), lambda i: (0, i, 0, 0)),
            pl.BlockSpec((1, BLK, 4 * C), lambda i: (i, 0, 0)),
            pl.BlockSpec((4 * C, 4 * C), lambda i: (0, 0)),
            pl.BlockSpec((4 * C, 4 * C), lambda i: (0, 0)),
            pl.BlockSpec((1, 4 * C), lambda i: (0, 0)),
        ],
        out_specs=pl.BlockSpec((1, BLK, 4 * C), lambda i: (i, 0, 0)),
        out_shape=jax.ShapeDtypeStruct((NGRID, BLK, 4 * C), jnp.float32),
    )(a, h, w1blk, w2blk, btile)


def _tc_final_body(a_ref, h_ref, w1_ref, w2_ref, b_ref, seg_ref, wd_ref,
                   bd_ref, o_ref, pool_acc, cnt_acc):
    i = pl.program_id(0)

    @pl.when(i == 0)
    def _init():
        pool_acc[...] = jnp.zeros_like(pool_acc)
        cnt_acc[...] = jnp.zeros_like(cnt_acc)

    agg = a_ref[0, 0] + a_ref[1, 0]
    h3 = (jnp.dot(agg, w1_ref[...], preferred_element_type=jnp.float32)
          + jnp.dot(h_ref[0], w2_ref[...], preferred_element_type=jnp.float32)
          + b_ref[...])
    h3 = jnp.maximum(h3, 0.0)

    dn = (((0,), (0,)), ((), ()))
    for u in range(4):
        onehot = (seg_ref[0][:, u:u + 1]
                  == lax.broadcasted_iota(jnp.int32, (BLK, B), 1))
        onehot = onehot.astype(jnp.float32)
        rows_u = h3[:, u * C:(u + 1) * C]
        pool_acc[...] += lax.dot_general(onehot, rows_u, dn,
                                         preferred_element_type=jnp.float32)
        cnt_acc[...] += lax.dot_general(
            onehot, jnp.ones((BLK, C), jnp.float32), dn,
            preferred_element_type=jnp.float32)

    @pl.when(i == NGRID - 1)
    def _finish():
        pooled = pool_acc[...] / jnp.maximum(cnt_acc[...], 1.0)
        z = jnp.dot(pooled, wd_ref[...],
                    preferred_element_type=jnp.float32) + bd_ref[...]
        o_ref[...] = jax.nn.sigmoid(z)


def _tc_final(a, h, w1blk, w2blk, btile, seg4, wd, bd):
    return pl.pallas_call(
        _tc_final_body,
        grid=(NGRID,),
        in_specs=[
            pl.BlockSpec((NC, 1, BLK, 4 * C), lambda i: (0, i, 0, 0)),
            pl.BlockSpec((1, BLK, 4 * C), lambda i: (i, 0, 0)),
            pl.BlockSpec((4 * C, 4 * C), lambda i: (0, 0)),
            pl.BlockSpec((4 * C, 4 * C), lambda i: (0, 0)),
            pl.BlockSpec((1, 4 * C), lambda i: (0, 0)),
            pl.BlockSpec((1, BLK, 4), lambda i: (i, 0, 0)),
            pl.BlockSpec((C, 1), lambda i: (0, 0)),
            pl.BlockSpec((1, 1), lambda i: (0, 0)),
        ],
        out_specs=pl.BlockSpec((B, 1), lambda i: (0, 0)),
        out_shape=jax.ShapeDtypeStruct((B, 1), jnp.float32),
        scratch_shapes=[
            pltpu.VMEM((B, C), jnp.float32),
            pltpu.VMEM((B, C), jnp.float32),
        ],
    )(a, h, w1blk, w2blk, btile, seg4, wd, bd)


def kernel(x, edge_index, edge_weight, segment_ids,
           W1_1, W2_1, b1, W1_2, W2_2, b2, W1_3, W2_3, b3, Wd, bd):
    src = edge_index[0].reshape(E // CH, CH)
    dst = edge_index[1].reshape(E // CH, CH)
    ew2 = edge_weight.reshape(E // CH, CH)
    x_flat = x.reshape(N)

    eye4 = jnp.eye(4, dtype=jnp.float32)
    pw1 = jnp.kron(eye4, W1_1)
    pw2 = jnp.kron(eye4, W2_1)
    w1blk2 = jnp.kron(eye4, W1_2)
    w2blk2 = jnp.kron(eye4, W2_2)
    w1blk3 = jnp.kron(eye4, W1_3)
    w2blk3 = jnp.kron(eye4, W2_3)

    s_parts = _sc_scalar_agg(x_flat, src, dst, ew2)
    h1p = _tc_dense1(s_parts.reshape(NC, NGRID, BLK, 4),
                     x.reshape(NGRID, BLK, 4),
                     pw1, pw2, jnp.tile(b1, 4).reshape(1, 4 * C))

    a2 = _sc_row_agg(h1p.reshape(N, C), src, dst, ew2)
    h2p = _tc_dense(a2.reshape(NC, NGRID, BLK, 4 * C), h1p, w1blk2, w2blk2,
                    jnp.tile(b2, 4).reshape(1, 4 * C))

    a3 = _sc_row_agg(h2p.reshape(N, C), src, dst, ew2)
    out = _tc_final(a3.reshape(NC, NGRID, BLK, 4 * C), h2p, w1blk3, w2blk3,
                    jnp.tile(b3, 4).reshape(1, 4 * C),
                    segment_ids.reshape(NGRID, BLK, 4), Wd, bd.reshape(1, 1))
    return out


# speedup vs baseline: 32.1044x; 1.0233x over previous
import functools

import jax
import jax.numpy as jnp
from jax import lax
from jax.experimental import pallas as pl
from jax.experimental.pallas import tpu as pltpu
from jax.experimental.pallas import tpu_sc as plsc

N = 50000
E = 1600000
B = 32
C = 32

NC = 2
NS = 16
NW = NC * NS
L = 16

TILE_E = E // NW
CH = 80
NSUB = 5
BE = CH * NSUB
NBLK = TILE_E // BE
NSUB1 = 25
BE1 = CH * NSUB1
NBLK1 = TILE_E // BE1

WR = 200

ACC1_PAD = 51200
Z1 = ACC1_PAD // NS
W1T = 10
WCH1 = N // W1T

NP = N // 4
BLK = 500
NGRID = NP // BLK

_mesh = plsc.VectorSubcoreMesh(core_axis_name="c", subcore_axis_name="s")


@functools.partial(
    pl.kernel,
    out_type=jax.ShapeDtypeStruct((NC * N,), jnp.float32),
    mesh=_mesh,
    scratch_types=[
        pltpu.VMEM((NSUB1, CH), jnp.int32),
        pltpu.VMEM((NSUB1, CH), jnp.int32),
        pltpu.VMEM((NSUB1, CH), jnp.float32),
        pltpu.VMEM((NSUB1, CH), jnp.float32),
        pltpu.VMEM((NSUB1, CH), jnp.int32),
        pltpu.VMEM((NSUB1, CH), jnp.int32),
        pltpu.VMEM((NSUB1, CH), jnp.float32),
        pltpu.VMEM((NSUB1, CH), jnp.float32),
        pltpu.VMEM((Z1,), jnp.float32),
        pltpu.VMEM((WCH1,), jnp.float32),
        pltpu.VMEM_SHARED((ACC1_PAD,), jnp.float32),
        pltpu.SemaphoreType.DMA,
        pltpu.SemaphoreType.DMA,
        pltpu.SemaphoreType.DMA,
        pltpu.SemaphoreType.DMA,
        pltpu.SemaphoreType.DMA,
        pltpu.SemaphoreType.DMA,
    ],
    compiler_params=pltpu.CompilerParams(use_tc_tiling_on_sc=False),
)
def _sc_scalar_agg(x_hbm, src_hbm, dst_hbm, ew_hbm, out_hbm,
                   srcbA, dstbA, ewbA, xbA, srcbB, dstbB, ewbB, xbB,
                   zbuf, wbuf, acc, isemA, isemB, gsemA, gsemB, ssemA, ssemB):
    cidx = lax.axis_index("c")
    sidx = lax.axis_index("s")
    zeros = jnp.zeros((L,), jnp.float32)

    @pl.loop(0, Z1 // L)
    def _zero_stage(i):
        zbuf[pl.ds(i * L, L)] = zeros

    pltpu.sync_copy(zbuf, acc.at[pl.ds(sidx * Z1, Z1)])
    plsc.subcore_barrier()

    row0 = (cidx * NS + sidx) * (TILE_E // CH)

    def fire_idx(rbase, sb, db, eb, sem):
        return [pltpu.async_copy(src_hbm.at[pl.ds(rbase, NSUB1)], sb, sem),
                pltpu.async_copy(dst_hbm.at[pl.ds(rbase, NSUB1)], db, sem),
                pltpu.async_copy(ew_hbm.at[pl.ds(rbase, NSUB1)], eb, sem)]

    def drain_idx(rbase, sb, db, eb, sem):
        pltpu.make_async_copy(src_hbm.at[pl.ds(rbase, NSUB1)], sb, sem).wait()
        pltpu.make_async_copy(dst_hbm.at[pl.ds(rbase, NSUB1)], db, sem).wait()
        pltpu.make_async_copy(ew_hbm.at[pl.ds(rbase, NSUB1)], eb, sem).wait()

    def fire_gathers(sb, xb, sem):
        return [pltpu.async_copy(x_hbm.at[sb.at[k]], xb.at[k], sem)
                for k in range(NSUB1)]

    def mult(xb, eb):
        @pl.loop(0, NSUB1)
        def _mul(k):
            for j in range(CH // L):
                sl = pl.ds(j * L, L)
                xb[k, sl] = xb[k, sl] * eb[k, sl]

    def fire_scatters(xb, db, sem):
        return [pltpu.async_copy(xb.at[k], acc.at[db.at[k]], sem, add=True)
                for k in range(NSUB1)]

    fire_idx(row0, srcbA, dstbA, ewbA, isemA)

    @pl.loop(0, (NBLK1 - 1) // 2)
    def _pipe(t):
        rbA = row0 + (2 * t) * NSUB1
        rbB = rbA + NSUB1
        drain_idx(rbA, srcbA, dstbA, ewbA, isemA)
        gA = fire_gathers(srcbA, xbA, gsemA)
        idxB = fire_idx(rbB, srcbB, dstbB, ewbB, isemB)
        for cp in gA:
            cp.wait()
        for cp in idxB:
            cp.wait()
        gB = fire_gathers(srcbB, xbB, gsemB)
        mult(xbA, ewbA)
        scatA = fire_scatters(xbA, dstbA, ssemA)
        for cp in gB:
            cp.wait()
        mult(xbB, ewbB)
        scatB = fire_scatters(xbB, dstbB, ssemB)
        fire_idx(rbA + 2 * NSUB1, srcbA, dstbA, ewbA, isemA)
        for cp in scatA:
            cp.wait()
        for cp in scatB:
            cp.wait()

    rbT = row0 + (NBLK1 - 1) * NSUB1
    drain_idx(rbT, srcbA, dstbA, ewbA, isemA)
    gT = fire_gathers(srcbA, xbA, gsemA)
    for cp in gT:
        cp.wait()
    mult(xbA, ewbA)
    scatT = fire_scatters(xbA, dstbA, ssemA)
    for cp in scatT:
        cp.wait()

    plsc.subcore_barrier()

    @pl.when(sidx < W1T)
    def _writeout():
        pltpu.sync_copy(acc.at[pl.ds(sidx * WCH1, WCH1)], wbuf)
        pltpu.sync_copy(wbuf, out_hbm.at[pl.ds(cidx * N + sidx * WCH1, WCH1)])


@functools.partial(
    pl.kernel,
    out_type=jax.ShapeDtypeStruct((NC, N, C), jnp.float32),
    mesh=_mesh,
    scratch_types=[
        pltpu.VMEM((NSUB, CH), jnp.int32),
        pltpu.VMEM((NSUB, CH), jnp.int32),
        pltpu.VMEM((NSUB, CH), jnp.float32),
        pltpu.VMEM((NSUB, CH), jnp.int32),
        pltpu.VMEM((NSUB, CH), jnp.int32),
        pltpu.VMEM((NSUB, CH), jnp.float32),
        pltpu.VMEM((BE, C), jnp.float32),
        pltpu.VMEM((BE, C), jnp.float32),
        pltpu.VMEM_SHARED((N, C), jnp.float32),
        pltpu.SemaphoreType.DMA,
        pltpu.SemaphoreType.DMA,
        [pltpu.SemaphoreType.DMA] * NSUB,
        [pltpu.SemaphoreType.DMA] * NSUB,
        pltpu.SemaphoreType.DMA,
        pltpu.SemaphoreType.DMA,
    ],
    compiler_params=pltpu.CompilerParams(use_tc_tiling_on_sc=False),
)
def _sc_row_agg(h_hbm, src_hbm, dst_hbm, ew_hbm, out_hbm,
                srcbA, dstbA, ewbA, srcbB, dstbB, ewbB, rowsA, rowsB, acc,
                isemA, isemB, gsemsA, gsemsB, ssemA, ssemB):
    cidx = lax.axis_index("c")
    sidx = lax.axis_index("s")
    zeros = jnp.zeros((L,), jnp.float32)
    gdn = lax.GatherDimensionNumbers(
        offset_dims=(), collapsed_slice_dims=(0,), start_index_map=(0,))

    @pl.loop(0, WR)
    def _zero_stage(r):
        rowsA[r, pl.ds(0, L)] = zeros
        rowsA[r, pl.ds(L, L)] = zeros

    @pl.when(sidx < W1T)
    def _zero_acc():
        @pl.loop(0, WCH1 // WR)
        def _(i):
            pltpu.sync_copy(rowsA.at[pl.ds(0, WR)],
                            acc.at[pl.ds(sidx * WCH1 + i * WR, WR)])

    plsc.subcore_barrier()

    row0 = (cidx * NS + sidx) * (TILE_E // CH)

    def fire_idx(rbase, sb, db, eb, sem):
        return [pltpu.async_copy(src_hbm.at[pl.ds(rbase, NSUB)], sb, sem),
                pltpu.async_copy(dst_hbm.at[pl.ds(rbase, NSUB)], db, sem),
                pltpu.async_copy(ew_hbm.at[pl.ds(rbase, NSUB)], eb, sem)]

    def drain_idx(rbase, sb, db, eb, sem):
        pltpu.make_async_copy(src_hbm.at[pl.ds(rbase, NSUB)], sb, sem).wait()
        pltpu.make_async_copy(dst_hbm.at[pl.ds(rbase, NSUB)], db, sem).wait()
        pltpu.make_async_copy(ew_hbm.at[pl.ds(rbase, NSUB)], eb, sem).wait()

    def fire_gathers(sb, rows, gsems):
        return [pltpu.async_copy(h_hbm.at[sb.at[k]],
                                 rows.at[pl.ds(k * CH, CH)], gsems[k])
                for k in range(NSUB)]

    def mult_scatter(k, eb, rows, db, ssem):
        @pl.loop(0, CH // L)
        def _scale(jj):
            wv = eb[k, pl.ds(jj * L, L)]
            for el in range(L):
                w = lax.gather(wv, jnp.full((L, 1), el, jnp.int32), gdn, (1,),
                               mode=lax.GatherScatterMode.PROMISE_IN_BOUNDS)
                e = k * CH + jj * L + el
                rows[e, pl.ds(0, L)] = rows[e, pl.ds(0, L)] * w
                rows[e, pl.ds(L, L)] = rows[e, pl.ds(L, L)] * w
        return pltpu.async_copy(rows.at[pl.ds(k * CH, CH)],
                                acc.at[db.at[k]], ssem, add=True)

    fire_idx(row0, srcbA, dstbA, ewbA, isemA)

    @pl.loop(0, (NBLK - 1) // 2)
    def _pipe(t):
        rbA = row0 + (2 * t) * NSUB
        rbB = rbA + NSUB
        drain_idx(rbA, srcbA, dstbA, ewbA, isemA)
        gA = fire_gathers(srcbA, rowsA, gsemsA)
        idxB = fire_idx(rbB, srcbB, dstbB, ewbB, isemB)
        scatA = []
        for k in (0, 1):
            gA[k].wait()
            scatA.append(mult_scatter(k, ewbA, rowsA, dstbA, ssemA))
        for cp in idxB:
            cp.wait()
        gB = fire_gathers(srcbB, rowsB, gsemsB)
        for k in (2, 3, 4):
            gA[k].wait()
            scatA.append(mult_scatter(k, ewbA, rowsA, dstbA, ssemA))
        for cp in scatA:
            cp.wait()
        scatB = []
        for k in range(NSUB):
            gB[k].wait()
            scatB.append(mult_scatter(k, ewbB, rowsB, dstbB, ssemB))
        fire_idx(rbA + 2 * NSUB, srcbA, dstbA, ewbA, isemA)
        for cp in scatB:
            cp.wait()

    rbT = row0 + (NBLK - 1) * NSUB
    drain_idx(rbT, srcbA, dstbA, ewbA, isemA)
    gT = fire_gathers(srcbA, rowsA, gsemsA)
    scatT = []
    for k in range(NSUB):
        gT[k].wait()
        scatT.append(mult_scatter(k, ewbA, rowsA, dstbA, ssemA))
    for cp in scatT:
        cp.wait()

    plsc.subcore_barrier()

    @pl.when(sidx < W1T)
    def _writeout():
        @pl.loop(0, WCH1 // WR)
        def _(i):
            r0 = sidx * WCH1 + i * WR
            pltpu.sync_copy(acc.at[pl.ds(r0, WR)], rowsA.at[pl.ds(0, WR)])
            pltpu.sync_copy(rowsA.at[pl.ds(0, WR)], out_hbm.at[cidx, pl.ds(r0, WR)])


def _tc_dense1_body(sp_ref, x_ref, pw1_ref, pw2_ref, b_ref, o_ref):
    s = sp_ref[0, 0] + sp_ref[1, 0]
    h = (jnp.dot(s, pw1_ref[...], preferred_element_type=jnp.float32)
         + jnp.dot(x_ref[0], pw2_ref[...],
                   preferred_element_type=jnp.float32)
         + b_ref[...])
    o_ref[0] = jnp.maximum(h, 0.0)


def _tc_dense1(s_parts, x4, pw1, pw2, btile):
    return pl.pallas_call(
        _tc_dense1_body,
        grid=(NGRID,),
        in_specs=[
            pl.BlockSpec((NC, 1, BLK, 4), lambda i: (0, i, 0, 0)),
            pl.BlockSpec((1, BLK, 4), lambda i: (i, 0, 0)),
            pl.BlockSpec((4, 4 * C), lambda i: (0, 0)),
            pl.BlockSpec((4, 4 * C), lambda i: (0, 0)),
            pl.BlockSpec((1, 4 * C), lambda i: (0, 0)),
        ],
        out_specs=pl.BlockSpec((1, BLK, 4 * C), lambda i: (i, 0, 0)),
        out_shape=jax.ShapeDtypeStruct((NGRID, BLK, 4 * C), jnp.float32),
    )(s_parts, x4, pw1, pw2, btile)


def _tc_dense_body(a_ref, h_ref, w1_ref, w2_ref, b_ref, o_ref):
    agg = a_ref[0, 0] + a_ref[1, 0]
    hn = (jnp.dot(agg, w1_ref[...], preferred_element_type=jnp.float32)
          + jnp.dot(h_ref[0], w2_ref[...], preferred_element_type=jnp.float32)
          + b_ref[...])
    o_ref[0] = jnp.maximum(hn, 0.0)


def _tc_dense(a, h, w1blk, w2blk, btile):
    return pl.pallas_call(
        _tc_dense_body,
        grid=(NGRID,),
        in_specs=[
            pl.BlockSpec((NC, 1, BLK, 4 * C), lambda i: (0, i, 0, 0)),
            pl.BlockSpec((1, BLK, 4 * C), lambda i: (i, 0, 0)),
            pl.BlockSpec((4 * C, 4 * C), lambda i: (0, 0)),
            pl.BlockSpec((4 * C, 4 * C), lambda i: (0, 0)),
            pl.BlockSpec((1, 4 * C), lambda i: (0, 0)),
        ],
        out_specs=pl.BlockSpec((1, BLK, 4 * C), lambda i: (i, 0, 0)),
        out_shape=jax.ShapeDtypeStruct((NGRID, BLK, 4 * C), jnp.float32),
    )(a, h, w1blk, w2blk, btile)


def _tc_final_body(a_ref, h_ref, w1_ref, w2_ref, b_ref, seg_ref, wd_ref,
                   bd_ref, o_ref, pool_acc, cnt_acc):
    i = pl.program_id(0)

    @pl.when(i == 0)
    def _init():
        pool_acc[...] = jnp.zeros_like(pool_acc)
        cnt_acc[...] = jnp.zeros_like(cnt_acc)

    agg = a_ref[0, 0] + a_ref[1, 0]
    h3 = (jnp.dot(agg, w1_ref[...], preferred_element_type=jnp.float32)
          + jnp.dot(h_ref[0], w2_ref[...], preferred_element_type=jnp.float32)
          + b_ref[...])
    h3 = jnp.maximum(h3, 0.0)

    dn = (((0,), (0,)), ((), ()))
    for u in range(4):
        onehot = (seg_ref[0][:, u:u + 1]
                  == lax.broadcasted_iota(jnp.int32, (BLK, B), 1))
        onehot = onehot.astype(jnp.float32)
        rows_u = h3[:, u * C:(u + 1) * C]
        pool_acc[...] += lax.dot_general(onehot, rows_u, dn,
                                         preferred_element_type=jnp.float32)
        cnt_acc[...] += lax.dot_general(
            onehot, jnp.ones((BLK, C), jnp.float32), dn,
            preferred_element_type=jnp.float32)

    @pl.when(i == NGRID - 1)
    def _finish():
        pooled = pool_acc[...] / jnp.maximum(cnt_acc[...], 1.0)
        z = jnp.dot(pooled, wd_ref[...],
                    preferred_element_type=jnp.float32) + bd_ref[...]
        o_ref[...] = jax.nn.sigmoid(z)


def _tc_final(a, h, w1blk, w2blk, btile, seg4, wd, bd):
    return pl.pallas_call(
        _tc_final_body,
        grid=(NGRID,),
        in_specs=[
            pl.BlockSpec((NC, 1, BLK, 4 * C), lambda i: (0, i, 0, 0)),
            pl.BlockSpec((1, BLK, 4 * C), lambda i: (i, 0, 0)),
            pl.BlockSpec((4 * C, 4 * C), lambda i: (0, 0)),
            pl.BlockSpec((4 * C, 4 * C), lambda i: (0, 0)),
            pl.BlockSpec((1, 4 * C), lambda i: (0, 0)),
            pl.BlockSpec((1, BLK, 4), lambda i: (i, 0, 0)),
            pl.BlockSpec((C, 1), lambda i: (0, 0)),
            pl.BlockSpec((1, 1), lambda i: (0, 0)),
        ],
        out_specs=pl.BlockSpec((B, 1), lambda i: (0, 0)),
        out_shape=jax.ShapeDtypeStruct((B, 1), jnp.float32),
        scratch_shapes=[
            pltpu.VMEM((B, C), jnp.float32),
            pltpu.VMEM((B, C), jnp.float32),
        ],
    )(a, h, w1blk, w2blk, btile, seg4, wd, bd)


def kernel(x, edge_index, edge_weight, segment_ids,
           W1_1, W2_1, b1, W1_2, W2_2, b2, W1_3, W2_3, b3, Wd, bd):
    src = edge_index[0].reshape(E // CH, CH)
    dst = edge_index[1].reshape(E // CH, CH)
    ew2 = edge_weight.reshape(E // CH, CH)
    x_flat = x.reshape(N)

    eye4 = jnp.eye(4, dtype=jnp.float32)
    pw1 = jnp.kron(eye4, W1_1)
    pw2 = jnp.kron(eye4, W2_1)
    w1blk2 = jnp.kron(eye4, W1_2)
    w2blk2 = jnp.kron(eye4, W2_2)
    w1blk3 = jnp.kron(eye4, W1_3)
    w2blk3 = jnp.kron(eye4, W2_3)

    s_parts = _sc_scalar_agg(x_flat, src, dst, ew2)
    h1p = _tc_dense1(s_parts.reshape(NC, NGRID, BLK, 4),
                     x.reshape(NGRID, BLK, 4),
                     pw1, pw2, jnp.tile(b1, 4).reshape(1, 4 * C))

    a2 = _sc_row_agg(h1p.reshape(N, C), src, dst, ew2)
    h2p = _tc_dense(a2.reshape(NC, NGRID, BLK, 4 * C), h1p, w1blk2, w2blk2,
                    jnp.tile(b2, 4).reshape(1, 4 * C))

    a3 = _sc_row_agg(h2p.reshape(N, C), src, dst, ew2)
    out = _tc_final(a3.reshape(NC, NGRID, BLK, 4 * C), h2p, w1blk3, w2blk3,
                    jnp.tile(b3, 4).reshape(1, 4 * C),
                    segment_ids.reshape(NGRID, BLK, 4), Wd, bd.reshape(1, 1))
    return out
